# baseline (device time: 72564 ns/iter reference)
import functools

import jax
import jax.numpy as jnp
from jax import lax
from jax.experimental import pallas as pl
from jax.experimental.pallas import tpu as pltpu

NEG = -1e30
PB = 8


def kernel(Q, K, V, bt, lens):
    B, _, H, D = Q.shape
    P_loc, BS = K.shape[0], K.shape[1]
    NB = bt.shape[1]
    n_steps = P_loc // PB
    scale = D ** -0.5

    my_x = lax.axis_index("x")

    slot_ok = jnp.arange(NB)[None, :] < lens[:, None]
    pid = jnp.where(slot_ok, bt, -1)
    pages = my_x * P_loc + jnp.arange(P_loc)
    C = (pid[:, :, None] == pages[None, None, :]).sum(1)
    Wk = jnp.repeat(C, BS, axis=1).astype(jnp.float32)

    qT = jnp.transpose(Q[:, 0], (1, 0, 2))

    def compute_body(q_ref, k_ref, v_ref, w_ref, acc_ref, stats_ref):
        p = pl.program_id(0)

        @pl.when(p == 0)
        def _():
            acc_ref[...] = jnp.zeros_like(acc_ref)
            stats_ref[0] = jnp.full((H, B), NEG, jnp.float32)
            stats_ref[1] = jnp.zeros((H, B), jnp.float32)

        q = q_ref[...]
        k = k_ref[...].reshape(PB * BS, H, D)
        v = v_ref[...].reshape(PB * BS, H, D)
        wk = w_ref[...]

        s = lax.dot_general(
            q, jnp.transpose(k, (1, 0, 2)),
            (((2,), (2,)), ((0,), (0,))),
            preferred_element_type=jnp.float32,
        ) * scale
        s = jnp.where((wk > 0)[None], s, NEG)

        m_old = stats_ref[0]
        l_old = stats_ref[1]
        m_new = jnp.maximum(m_old, s.max(-1))
        alpha = jnp.exp(m_old - m_new)
        e = jnp.exp(s - m_new[:, :, None]) * wk[None]
        l_new = l_old * alpha + e.sum(-1)
        pv = lax.dot_general(
            e, jnp.transpose(v, (1, 0, 2)),
            (((2,), (1,)), ((0,), (0,))),
            preferred_element_type=jnp.float32,
        )
        acc_ref[...] = acc_ref[...] * alpha[:, :, None] + pv
        stats_ref[0] = m_new
        stats_ref[1] = l_new

    acc, stats = pl.pallas_call(
        compute_body,
        grid=(n_steps,),
        in_specs=[
            pl.BlockSpec((H, B, D), lambda p: (0, 0, 0)),
            pl.BlockSpec((PB, BS, H, D), lambda p: (p, 0, 0, 0)),
            pl.BlockSpec((PB, BS, H, D), lambda p: (p, 0, 0, 0)),
            pl.BlockSpec((B, PB * BS), lambda p: (0, p)),
        ],
        out_specs=[
            pl.BlockSpec((H, B, D), lambda p: (0, 0, 0)),
            pl.BlockSpec((2, H, B), lambda p: (0, 0, 0)),
        ],
        out_shape=[
            jax.ShapeDtypeStruct((H, B, D), jnp.float32),
            jax.ShapeDtypeStruct((2, H, B), jnp.float32),
        ],
    )(qT, K, V, Wk)

    def merge_body(acc_ref, stats_ref, out_ref,
                   racc_ref, rstats_ref, send_sems, recv_sems):
        mx = lax.axis_index("x")
        my = lax.axis_index("y")
        mz = lax.axis_index("z")
        nbr = (1 - mx, my, mz)

        barrier = pltpu.get_barrier_semaphore()
        pl.semaphore_signal(barrier, inc=1, device_id=nbr,
                            device_id_type=pl.DeviceIdType.MESH)
        pl.semaphore_wait(barrier, 1)

        acc_rdma = pltpu.make_async_remote_copy(
            src_ref=acc_ref, dst_ref=racc_ref,
            send_sem=send_sems.at[0], recv_sem=recv_sems.at[0],
            device_id=nbr, device_id_type=pl.DeviceIdType.MESH)
        st_rdma = pltpu.make_async_remote_copy(
            src_ref=stats_ref, dst_ref=rstats_ref,
            send_sem=send_sems.at[1], recv_sem=recv_sems.at[1],
            device_id=nbr, device_id_type=pl.DeviceIdType.MESH)
        acc_rdma.start()
        st_rdma.start()
        acc_rdma.wait()
        st_rdma.wait()

        m1, l1 = stats_ref[0], stats_ref[1]
        m2, l2 = rstats_ref[0], rstats_ref[1]
        M = jnp.maximum(m1, m2)
        a1 = jnp.exp(m1 - M)
        a2 = jnp.exp(m2 - M)
        Lsum = a1 * l1 + a2 * l2
        num = acc_ref[...] * a1[:, :, None] + racc_ref[...] * a2[:, :, None]
        o = num / Lsum[:, :, None]
        out_ref[...] = jnp.transpose(o, (1, 0, 2))[:, None]

        @functools.partial(pl.run_scoped, exit_sem=pltpu.SemaphoreType.REGULAR)
        def _(exit_sem):
            pl.semaphore_signal(exit_sem, inc=1, device_id=nbr,
                                device_id_type=pl.DeviceIdType.MESH)
            pl.semaphore_wait(exit_sem, 1)

    out = pl.pallas_call(
        merge_body,
        out_shape=jax.ShapeDtypeStruct((B, 1, H, D), jnp.float32),
        in_specs=[
            pl.BlockSpec(memory_space=pltpu.VMEM),
            pl.BlockSpec(memory_space=pltpu.VMEM),
        ],
        out_specs=pl.BlockSpec(memory_space=pltpu.VMEM),
        scratch_shapes=[
            pltpu.VMEM((H, B, D), jnp.float32),
            pltpu.VMEM((2, H, B), jnp.float32),
            pltpu.SemaphoreType.DMA((2,)),
            pltpu.SemaphoreType.DMA((2,)),
        ],
        compiler_params=pltpu.CompilerParams(collective_id=0),
    )(acc, stats)
    return out


# device time: 38587 ns/iter; 1.8805x vs baseline; 1.8805x over previous
import functools

import jax
import jax.numpy as jnp
from jax import lax
from jax.experimental import pallas as pl
from jax.experimental.pallas import tpu as pltpu

NEG = -1e30
PB = 8


def kernel(Q, K, V, bt, lens):
    B, _, H, D = Q.shape
    P_loc, BS = K.shape[0], K.shape[1]
    NB = bt.shape[1]
    QP = P_loc // 4
    n_steps = QP // PB
    scale = D ** -0.5

    mx = lax.axis_index("x")
    my = lax.axis_index("y")
    mz = lax.axis_index("z")
    quarter = my * 2 + mz

    slot_ok = jnp.arange(NB)[None, :] < lens[:, None]
    pid = jnp.where(slot_ok, bt, -1)
    pages = mx * P_loc + quarter * QP + jnp.arange(QP)
    C = (pid[:, :, None] == pages[None, None, :]).sum(1)
    Wk = jnp.repeat(C, BS, axis=1).astype(jnp.float32)

    qT = jnp.transpose(Q[:, 0], (1, 0, 2))
    qbase = jnp.full((1,), quarter * n_steps, jnp.int32)

    def compute_body(qb_ref, q_ref, k_ref, v_ref, w_ref, acc_ref, stats_ref):
        p = pl.program_id(0)

        @pl.when(p == 0)
        def _():
            acc_ref[...] = jnp.zeros_like(acc_ref)
            stats_ref[0] = jnp.full((H, B), NEG, jnp.float32)
            stats_ref[1] = jnp.zeros((H, B), jnp.float32)

        q = q_ref[...]
        k = k_ref[...].reshape(PB * BS, H, D)
        v = v_ref[...].reshape(PB * BS, H, D)
        wk = w_ref[...]

        s = lax.dot_general(
            q, jnp.transpose(k, (1, 0, 2)),
            (((2,), (2,)), ((0,), (0,))),
            preferred_element_type=jnp.float32,
        ) * scale
        s = jnp.where((wk > 0)[None], s, NEG)

        m_old = stats_ref[0]
        l_old = stats_ref[1]
        m_new = jnp.maximum(m_old, s.max(-1))
        alpha = jnp.exp(m_old - m_new)
        e = jnp.exp(s - m_new[:, :, None]) * wk[None]
        l_new = l_old * alpha + e.sum(-1)
        pv = lax.dot_general(
            e, jnp.transpose(v, (1, 0, 2)),
            (((2,), (1,)), ((0,), (0,))),
            preferred_element_type=jnp.float32,
        )
        acc_ref[...] = acc_ref[...] * alpha[:, :, None] + pv
        stats_ref[0] = m_new
        stats_ref[1] = l_new

    grid_spec = pltpu.PrefetchScalarGridSpec(
        num_scalar_prefetch=1,
        grid=(n_steps,),
        in_specs=[
            pl.BlockSpec((H, B, D), lambda p, qb: (0, 0, 0)),
            pl.BlockSpec((PB, BS, H, D), lambda p, qb: (qb[0] + p, 0, 0, 0)),
            pl.BlockSpec((PB, BS, H, D), lambda p, qb: (qb[0] + p, 0, 0, 0)),
            pl.BlockSpec((B, PB * BS), lambda p, qb: (0, p)),
        ],
        out_specs=[
            pl.BlockSpec((H, B, D), lambda p, qb: (0, 0, 0)),
            pl.BlockSpec((2, H, B), lambda p, qb: (0, 0, 0)),
        ],
    )
    acc, stats = pl.pallas_call(
        compute_body,
        grid_spec=grid_spec,
        out_shape=[
            jax.ShapeDtypeStruct((H, B, D), jnp.float32),
            jax.ShapeDtypeStruct((2, H, B), jnp.float32),
        ],
    )(qbase, qT, K, V, Wk)

    def merge_body(acc_ref, stats_ref, out_ref, racc_ref, rstats_ref,
                   sacc_ref, sstats_ref, asend, arecv, ssend, srecv):
        mx_ = lax.axis_index("x")
        my_ = lax.axis_index("y")
        mz_ = lax.axis_index("z")
        nbrs = [(mx_, my_, 1 - mz_),
                (mx_, 1 - my_, mz_),
                (1 - mx_, my_, mz_)]

        barrier = pltpu.get_barrier_semaphore()
        for nb in nbrs:
            pl.semaphore_signal(barrier, inc=1, device_id=nb,
                                device_id_type=pl.DeviceIdType.MESH)
        pl.semaphore_wait(barrier, 3)

        macc = acc_ref[...]
        mm = stats_ref[0]
        ml = stats_ref[1]
        for s, nb in enumerate(nbrs):
            src_acc = acc_ref if s == 0 else sacc_ref
            src_st = stats_ref if s == 0 else sstats_ref
            a_rdma = pltpu.make_async_remote_copy(
                src_ref=src_acc, dst_ref=racc_ref.at[s],
                send_sem=asend.at[s], recv_sem=arecv.at[s],
                device_id=nb, device_id_type=pl.DeviceIdType.MESH)
            s_rdma = pltpu.make_async_remote_copy(
                src_ref=src_st, dst_ref=rstats_ref.at[s],
                send_sem=ssend.at[s], recv_sem=srecv.at[s],
                device_id=nb, device_id_type=pl.DeviceIdType.MESH)
            a_rdma.start()
            s_rdma.start()
            a_rdma.wait()
            s_rdma.wait()

            m2 = rstats_ref[s, 0]
            l2 = rstats_ref[s, 1]
            Mx = jnp.maximum(mm, m2)
            a1 = jnp.exp(mm - Mx)
            a2 = jnp.exp(m2 - Mx)
            macc = macc * a1[:, :, None] + racc_ref[s] * a2[:, :, None]
            ml = ml * a1 + l2 * a2
            mm = Mx
            if s < 2:
                sacc_ref[...] = macc
                sstats_ref[0] = mm
                sstats_ref[1] = ml

        o = macc / ml[:, :, None]
        out_ref[...] = jnp.transpose(o, (1, 0, 2))[:, None]

        @functools.partial(pl.run_scoped, exit_sem=pltpu.SemaphoreType.REGULAR)
        def _(exit_sem):
            for nb in nbrs:
                pl.semaphore_signal(exit_sem, inc=1, device_id=nb,
                                    device_id_type=pl.DeviceIdType.MESH)
            pl.semaphore_wait(exit_sem, 3)

    out = pl.pallas_call(
        merge_body,
        out_shape=jax.ShapeDtypeStruct((B, 1, H, D), jnp.float32),
        in_specs=[
            pl.BlockSpec(memory_space=pltpu.VMEM),
            pl.BlockSpec(memory_space=pltpu.VMEM),
        ],
        out_specs=pl.BlockSpec(memory_space=pltpu.VMEM),
        scratch_shapes=[
            pltpu.VMEM((3, H, B, D), jnp.float32),
            pltpu.VMEM((3, 2, H, B), jnp.float32),
            pltpu.VMEM((H, B, D), jnp.float32),
            pltpu.VMEM((2, H, B), jnp.float32),
            pltpu.SemaphoreType.DMA((3,)),
            pltpu.SemaphoreType.DMA((3,)),
            pltpu.SemaphoreType.DMA((3,)),
            pltpu.SemaphoreType.DMA((3,)),
        ],
        compiler_params=pltpu.CompilerParams(collective_id=0),
    )(acc, stats)
    return out


# device time: 34413 ns/iter; 2.1086x vs baseline; 1.1213x over previous
import functools

import jax
import jax.numpy as jnp
from jax import lax
from jax.experimental import pallas as pl
from jax.experimental.pallas import tpu as pltpu

NEG = -1e30
PB = 8


def kernel(Q, K, V, bt, lens):
    B, _, H, D = Q.shape
    P_loc, BS = K.shape[0], K.shape[1]
    NB = bt.shape[1]
    QP = P_loc // 4
    n_steps = QP // PB
    scale = D ** -0.5

    mx = lax.axis_index("x")
    my = lax.axis_index("y")
    mz = lax.axis_index("z")
    quarter = my * 2 + mz

    slot_ok = jnp.arange(NB)[None, :] < lens[:, None]
    pid = jnp.where(slot_ok, bt, -1)
    pages = mx * P_loc + quarter * QP + jnp.arange(QP)
    C = (pid[:, :, None] == pages[None, None, :]).sum(1)
    Wk = jnp.repeat(C, BS, axis=1).astype(jnp.float32)

    qT = jnp.transpose(Q[:, 0], (1, 0, 2))
    qbase = jnp.full((1,), quarter * n_steps, jnp.int32)

    def compute_body(qb_ref, q_ref, k_ref, v_ref, w_ref, acc_ref, stats_ref):
        p = pl.program_id(0)

        @pl.when(p == 0)
        def _():
            acc_ref[...] = jnp.zeros_like(acc_ref)
            stats_ref[0] = jnp.full((H, B), NEG, jnp.float32)
            stats_ref[1] = jnp.zeros((H, B), jnp.float32)

        q = q_ref[...]
        k = k_ref[...].reshape(PB * BS, H, D)
        v = v_ref[...].reshape(PB * BS, H, D)
        wk = w_ref[...]

        s = lax.dot_general(
            q, jnp.transpose(k, (1, 0, 2)),
            (((2,), (2,)), ((0,), (0,))),
            preferred_element_type=jnp.float32,
        ) * scale
        s = jnp.where((wk > 0)[None], s, NEG)

        m_old = stats_ref[0]
        l_old = stats_ref[1]
        m_new = jnp.maximum(m_old, s.max(-1))
        alpha = jnp.exp(m_old - m_new)
        e = jnp.exp(s - m_new[:, :, None]) * wk[None]
        l_new = l_old * alpha + e.sum(-1)
        pv = lax.dot_general(
            e, jnp.transpose(v, (1, 0, 2)),
            (((2,), (1,)), ((0,), (0,))),
            preferred_element_type=jnp.float32,
        )
        acc_ref[...] = acc_ref[...] * alpha[:, :, None] + pv
        stats_ref[0] = m_new
        stats_ref[1] = l_new

    grid_spec = pltpu.PrefetchScalarGridSpec(
        num_scalar_prefetch=1,
        grid=(n_steps,),
        in_specs=[
            pl.BlockSpec((H, B, D), lambda p, qb: (0, 0, 0)),
            pl.BlockSpec((PB, BS, H, D), lambda p, qb: (qb[0] + p, 0, 0, 0)),
            pl.BlockSpec((PB, BS, H, D), lambda p, qb: (qb[0] + p, 0, 0, 0)),
            pl.BlockSpec((B, PB * BS), lambda p, qb: (0, p)),
        ],
        out_specs=[
            pl.BlockSpec((H, B, D), lambda p, qb: (0, 0, 0)),
            pl.BlockSpec((2, H, B), lambda p, qb: (0, 0, 0)),
        ],
    )
    acc, stats = pl.pallas_call(
        compute_body,
        grid_spec=grid_spec,
        out_shape=[
            jax.ShapeDtypeStruct((H, B, D), jnp.float32),
            jax.ShapeDtypeStruct((2, H, B), jnp.float32),
        ],
    )(qbase, qT, K, V, Wk)

    def merge_body(acc_ref, stats_ref, out_ref, racc_ref, rstats_ref,
                   sacc_ref, sstats_ref, asend, arecv, ssend, srecv):
        mx_ = lax.axis_index("x")
        my_ = lax.axis_index("y")
        mz_ = lax.axis_index("z")
        nbrs = [(mx_, my_, 1 - mz_),
                (mx_, 1 - my_, mz_),
                (1 - mx_, my_, mz_)]

        barrier = pltpu.get_barrier_semaphore()
        for nb in nbrs:
            pl.semaphore_signal(barrier, inc=1, device_id=nb,
                                device_id_type=pl.DeviceIdType.MESH)
        pl.semaphore_wait(barrier, 3)

        macc = acc_ref[...]
        mm = stats_ref[0]
        ml = stats_ref[1]
        sacc_ref[...] = macc.astype(jnp.bfloat16)
        sstats_ref[0] = mm
        sstats_ref[1] = ml
        for s, nb in enumerate(nbrs):
            a_rdma = pltpu.make_async_remote_copy(
                src_ref=sacc_ref, dst_ref=racc_ref.at[s],
                send_sem=asend.at[s], recv_sem=arecv.at[s],
                device_id=nb, device_id_type=pl.DeviceIdType.MESH)
            s_rdma = pltpu.make_async_remote_copy(
                src_ref=sstats_ref, dst_ref=rstats_ref.at[s],
                send_sem=ssend.at[s], recv_sem=srecv.at[s],
                device_id=nb, device_id_type=pl.DeviceIdType.MESH)
            a_rdma.start()
            s_rdma.start()
            a_rdma.wait()
            s_rdma.wait()

            m2 = rstats_ref[s, 0]
            l2 = rstats_ref[s, 1]
            Mx = jnp.maximum(mm, m2)
            a1 = jnp.exp(mm - Mx)
            a2 = jnp.exp(m2 - Mx)
            racc = racc_ref[s].astype(jnp.float32)
            macc = macc * a1[:, :, None] + racc * a2[:, :, None]
            ml = ml * a1 + l2 * a2
            mm = Mx
            if s < 2:
                sacc_ref[...] = macc.astype(jnp.bfloat16)
                sstats_ref[0] = mm
                sstats_ref[1] = ml

        o = macc / ml[:, :, None]
        out_ref[...] = jnp.transpose(o, (1, 0, 2))[:, None]

        @functools.partial(pl.run_scoped, exit_sem=pltpu.SemaphoreType.REGULAR)
        def _(exit_sem):
            for nb in nbrs:
                pl.semaphore_signal(exit_sem, inc=1, device_id=nb,
                                    device_id_type=pl.DeviceIdType.MESH)
            pl.semaphore_wait(exit_sem, 3)

    out = pl.pallas_call(
        merge_body,
        out_shape=jax.ShapeDtypeStruct((B, 1, H, D), jnp.float32),
        in_specs=[
            pl.BlockSpec(memory_space=pltpu.VMEM),
            pl.BlockSpec(memory_space=pltpu.VMEM),
        ],
        out_specs=pl.BlockSpec(memory_space=pltpu.VMEM),
        scratch_shapes=[
            pltpu.VMEM((3, H, B, D), jnp.bfloat16),
            pltpu.VMEM((3, 2, H, B), jnp.float32),
            pltpu.VMEM((H, B, D), jnp.bfloat16),
            pltpu.VMEM((2, H, B), jnp.float32),
            pltpu.SemaphoreType.DMA((3,)),
            pltpu.SemaphoreType.DMA((3,)),
            pltpu.SemaphoreType.DMA((3,)),
            pltpu.SemaphoreType.DMA((3,)),
        ],
        compiler_params=pltpu.CompilerParams(collective_id=0),
    )(acc, stats)
    return out


# device time: 33565 ns/iter; 2.1619x vs baseline; 1.0253x over previous
import functools

import jax
import jax.numpy as jnp
from jax import lax
from jax.experimental import pallas as pl
from jax.experimental.pallas import tpu as pltpu

NEG = -1e30
PB = 8


def kernel(Q, K, V, bt, lens):
    B, _, H, D = Q.shape
    P_loc, BS = K.shape[0], K.shape[1]
    NB = bt.shape[1]
    QP = P_loc // 4
    n_steps = QP // PB
    HALF = n_steps // 2
    scale = D ** -0.5

    mx = lax.axis_index("x")
    my = lax.axis_index("y")
    mz = lax.axis_index("z")
    quarter = my * 2 + mz

    slot_ok = jnp.arange(NB)[None, :] < lens[:, None]
    pid = jnp.where(slot_ok, bt, -1)
    pages = mx * P_loc + quarter * QP + jnp.arange(QP)
    C = (pid[:, :, None] == pages[None, None, :]).sum(1)
    Wk = jnp.repeat(C, BS, axis=1).astype(jnp.float32)

    qT = jnp.transpose(Q[:, 0], (1, 0, 2))
    qbase = jnp.full((1,), quarter * n_steps, jnp.int32)

    def body(qb_ref, q_ref, k_ref, v_ref, w_ref, out_ref,
             acc_ref, stats_ref, sacc, sst, racc, rst,
             asend, arecv, ssend, srecv):
        p = pl.program_id(0)
        mx_ = lax.axis_index("x")
        my_ = lax.axis_index("y")
        mz_ = lax.axis_index("z")
        nb_z = (mx_, my_, 1 - mz_)
        nb_y = (mx_, 1 - my_, mz_)
        nb_x = (1 - mx_, my_, mz_)
        nbrs = [nb_z, nb_y, nb_x]

        def acc_rdma(slot, nb):
            return pltpu.make_async_remote_copy(
                src_ref=sacc.at[slot], dst_ref=racc.at[slot],
                send_sem=asend.at[slot], recv_sem=arecv.at[slot],
                device_id=nb, device_id_type=pl.DeviceIdType.MESH)

        def st_rdma(slot, nb):
            return pltpu.make_async_remote_copy(
                src_ref=sst.at[slot], dst_ref=rst.at[slot],
                send_sem=ssend.at[slot], recv_sem=srecv.at[slot],
                device_id=nb, device_id_type=pl.DeviceIdType.MESH)

        def reset_state():
            acc_ref[...] = jnp.zeros_like(acc_ref)
            stats_ref[0] = jnp.full((H, B), NEG, jnp.float32)
            stats_ref[1] = jnp.zeros((H, B), jnp.float32)

        @pl.when(p == 0)
        def _():
            barrier = pltpu.get_barrier_semaphore()
            for nb in nbrs:
                pl.semaphore_signal(barrier, inc=1, device_id=nb,
                                    device_id_type=pl.DeviceIdType.MESH)
            pl.semaphore_wait(barrier, 3)
            reset_state()

        q = q_ref[...]
        k = k_ref[...].reshape(PB * BS, H, D)
        v = v_ref[...].reshape(PB * BS, H, D)
        wk = w_ref[...]

        s = lax.dot_general(
            q, jnp.transpose(k, (1, 0, 2)),
            (((2,), (2,)), ((0,), (0,))),
            preferred_element_type=jnp.float32,
        ) * scale
        s = jnp.where((wk > 0)[None], s, NEG)

        m_old = stats_ref[0]
        l_old = stats_ref[1]
        m_new = jnp.maximum(m_old, s.max(-1))
        alpha = jnp.exp(m_old - m_new)
        e = jnp.exp(s - m_new[:, :, None]) * wk[None]
        l_new = l_old * alpha + e.sum(-1)
        pv = lax.dot_general(
            e, jnp.transpose(v, (1, 0, 2)),
            (((2,), (1,)), ((0,), (0,))),
            preferred_element_type=jnp.float32,
        )
        acc_ref[...] = acc_ref[...] * alpha[:, :, None] + pv
        stats_ref[0] = m_new
        stats_ref[1] = l_new

        @pl.when(p == HALF - 1)
        def _():
            sacc[0] = acc_ref[...].astype(jnp.bfloat16)
            sst[0] = stats_ref[...]
            acc_rdma(0, nb_z).start()
            st_rdma(0, nb_z).start()
            reset_state()

        @pl.when(p == n_steps - 1)
        def _():
            sacc[1] = acc_ref[...].astype(jnp.bfloat16)
            sst[1] = stats_ref[...]
            acc_rdma(1, nb_z).start()
            st_rdma(1, nb_z).start()

            acc_rdma(0, nb_z).wait()
            st_rdma(0, nb_z).wait()
            acc_rdma(1, nb_z).wait()
            st_rdma(1, nb_z).wait()

            def merge(macc, mm, ml, oacc, om, ol):
                Mx = jnp.maximum(mm, om)
                a1 = jnp.exp(mm - Mx)
                a2 = jnp.exp(om - Mx)
                return (macc * a1[:, :, None] + oacc * a2[:, :, None],
                        Mx, ml * a1 + ol * a2)

            macc = sacc[0].astype(jnp.float32)
            mm, ml = sst[0, 0], sst[0, 1]
            macc, mm, ml = merge(macc, mm, ml,
                                 acc_ref[...], stats_ref[0], stats_ref[1])
            macc, mm, ml = merge(macc, mm, ml,
                                 racc[0].astype(jnp.float32),
                                 rst[0, 0], rst[0, 1])
            macc, mm, ml = merge(macc, mm, ml,
                                 racc[1].astype(jnp.float32),
                                 rst[1, 0], rst[1, 1])

            for slot, nb in ((2, nb_y), (3, nb_x)):
                sacc[slot] = macc.astype(jnp.bfloat16)
                sst[slot, 0] = mm
                sst[slot, 1] = ml
                acc_rdma(slot, nb).start()
                st_rdma(slot, nb).start()
                acc_rdma(slot, nb).wait()
                st_rdma(slot, nb).wait()
                macc, mm, ml = merge(macc, mm, ml,
                                     racc[slot].astype(jnp.float32),
                                     rst[slot, 0], rst[slot, 1])

            o = macc / ml[:, :, None]
            out_ref[...] = jnp.transpose(o, (1, 0, 2))[:, None]

            @functools.partial(pl.run_scoped,
                               exit_sem=pltpu.SemaphoreType.REGULAR)
            def _(exit_sem):
                for nb in nbrs:
                    pl.semaphore_signal(exit_sem, inc=1, device_id=nb,
                                        device_id_type=pl.DeviceIdType.MESH)
                pl.semaphore_wait(exit_sem, 3)

    grid_spec = pltpu.PrefetchScalarGridSpec(
        num_scalar_prefetch=1,
        grid=(n_steps,),
        in_specs=[
            pl.BlockSpec((H, B, D), lambda p, qb: (0, 0, 0)),
            pl.BlockSpec((PB, BS, H, D), lambda p, qb: (qb[0] + p, 0, 0, 0)),
            pl.BlockSpec((PB, BS, H, D), lambda p, qb: (qb[0] + p, 0, 0, 0)),
            pl.BlockSpec((B, PB * BS), lambda p, qb: (0, p)),
        ],
        out_specs=pl.BlockSpec((B, 1, H, D), lambda p, qb: (0, 0, 0, 0)),
        scratch_shapes=[
            pltpu.VMEM((H, B, D), jnp.float32),
            pltpu.VMEM((2, H, B), jnp.float32),
            pltpu.VMEM((4, H, B, D), jnp.bfloat16),
            pltpu.VMEM((4, 2, H, B), jnp.float32),
            pltpu.VMEM((4, H, B, D), jnp.bfloat16),
            pltpu.VMEM((4, 2, H, B), jnp.float32),
            pltpu.SemaphoreType.DMA((4,)),
            pltpu.SemaphoreType.DMA((4,)),
            pltpu.SemaphoreType.DMA((4,)),
            pltpu.SemaphoreType.DMA((4,)),
        ],
    )
    out = pl.pallas_call(
        body,
        grid_spec=grid_spec,
        out_shape=jax.ShapeDtypeStruct((B, 1, H, D), jnp.float32),
        compiler_params=pltpu.CompilerParams(collective_id=0),
    )(qbase, qT, K, V, Wk)
    return out
